# baseline (speedup 1.0000x reference)
import jax
import jax.numpy as jnp
from jax.experimental import pallas as pl
from jax.experimental.pallas import tpu as pltpu

_BM = 400
_BM2 = 200


def _bf16_mm(a, b, dims):
    return jax.lax.dot_general(
        a.astype(jnp.bfloat16), b.astype(jnp.bfloat16), (dims, ((), ())),
        preferred_element_type=jnp.float32)


def _intra_partial(za_ref, i, bm, c0, c1):
    z = za_ref[pl.ds(i * bm, bm), :]
    sim = _bf16_mm(z, za_ref[c0:c1, :], ((1,), (1,)))
    return jnp.sum(jnp.exp(sim), axis=1, keepdims=True)


def _gcn_a_body(adj_ref, bf1_ref, bf2_ref, w1_ref, w2_ref,
                b_ref, a_ref, za_ref, sf2_ref, sf1_ref):
    @pl.when(pl.program_id(0) == 0)
    def _():
        sf1_ref[...] = jax.lax.dot_general(
            bf1_ref[...], w1_ref[...], (((1,), (1,)), ((), ())),
            preferred_element_type=jnp.float32)
        sf2_ref[...] = jax.lax.dot_general(
            bf2_ref[...], w2_ref[...], (((1,), (1,)), ((), ())),
            preferred_element_type=jnp.float32)

    acc = _bf16_mm(adj_ref[...], sf1_ref[...], ((1,), (0,)))
    out = acc + b_ref[...]
    a = a_ref[0, 0]
    out = jnp.where(out >= 0, out, a * out)
    nrm = jnp.sqrt(jnp.sum(out * out, axis=1, keepdims=True))
    za_ref[...] = out / jnp.maximum(nrm, 1e-12)


def _fused_b_body(d_ref, g_ref, sf_ref, b_ref, a_ref, za_ref, out_ref,
                  bemb_ref, h_ref, s1_ref, s2_ref):
    s = pl.program_id(0)
    nblk = pl.num_programs(0) // 3
    n, nh = h_ref.shape
    bm = d_ref.shape[0]

    @pl.when(s < nblk)
    def _():
        i = s
        acc = _bf16_mm(d_ref[...], sf_ref[...], ((1,), (0,)))
        out = acc + b_ref[...]
        a = a_ref[0, 0]
        bemb_ref[pl.ds(i * bm, bm), :] = jnp.where(out >= 0, out, a * out)
        s1_ref[pl.ds(i * bm, bm), :] = _intra_partial(za_ref, i, bm, 0, n // 2)

    @pl.when(jnp.logical_and(s >= nblk, s < 2 * nblk))
    def _():
        i = s - nblk
        h_ref[pl.ds(i * bm, bm), :] = _bf16_mm(
            g_ref[...], bemb_ref[...], ((1,), (0,)))
        s2_ref[pl.ds(i * bm, bm), :] = _intra_partial(za_ref, i, bm, n // 2, n)

    @pl.when(s >= 2 * nblk)
    def _():
        i = s - 2 * nblk
        nb = _bf16_mm(g_ref[...], h_ref[...], ((1,), (0,)))
        nrm = jnp.sqrt(jnp.sum(nb * nb, axis=1, keepdims=True))
        znb = nb / jnp.maximum(nrm, 1e-12)
        z = za_ref[pl.ds(i * bm, bm), :]
        inter = jnp.sum(z * znb, axis=1)
        diag = jnp.sum(z * z, axis=1)
        stot = s1_ref[pl.ds(i * bm, bm), 0] + s2_ref[pl.ds(i * bm, bm), 0]
        li = -inter + jnp.log(stot - jnp.exp(diag))

        @pl.when(s == 2 * nblk)
        def _():
            out_ref[...] = jnp.zeros((1, 1), jnp.float32)

        out_ref[...] += jnp.sum(li).reshape(1, 1)


def _gcn_b_plain_body(adj_ref, sf_ref, b_ref, a_ref, out_ref):
    acc = _bf16_mm(adj_ref[...], sf_ref[...], ((1,), (0,)))
    out = acc + b_ref[...]
    a = a_ref[0, 0]
    out_ref[...] = jnp.where(out >= 0, out, a * out)


def _hop_body(adj_ref, x_ref, out_ref):
    out_ref[...] = _bf16_mm(adj_ref[...], x_ref[...], ((1,), (0,)))


def _loss_tail_body(za_blk_ref, nb_ref, za_ref, out_ref):
    i = pl.program_id(0)
    n = za_ref.shape[0]
    bm = za_blk_ref.shape[0]
    z = za_blk_ref[...]
    nb = nb_ref[...]
    nrm = jnp.sqrt(jnp.sum(nb * nb, axis=1, keepdims=True))
    znb = nb / jnp.maximum(nrm, 1e-12)
    inter = jnp.sum(z * znb, axis=1)
    diag = jnp.sum(z * z, axis=1)
    stot = _intra_partial(za_ref, i, bm, 0, n)[:, 0]
    li = -inter + jnp.log(stot - jnp.exp(diag))

    @pl.when(i == 0)
    def _():
        out_ref[...] = jnp.zeros((1, 1), jnp.float32)

    out_ref[...] += jnp.sum(li).reshape(1, 1)


def kernel(bf1, bf2, ba, bd, adj, W1, b1, a1, W2, b2, a2, num_hop, sparse):
    n = ba.shape[-1]
    nin = bf1.shape[-1]
    nh = W1.shape[0]
    bm = _BM
    nblk = n // bm
    bm2 = _BM2
    nblk2 = n // bm2

    x1 = bf1.reshape(n, nin)
    x2 = bf2.reshape(n, nin)
    A = ba.reshape(n, n)
    D = bd.reshape(n, n)
    G = adj.reshape(n, n)
    b1r = b1.reshape(1, nh)
    b2r = b2.reshape(1, nh)
    a1r = a1.reshape(1, 1)
    a2r = a2.reshape(1, 1)

    adj_spec = pl.BlockSpec((bm, n), lambda i: (i, 0))
    full_spec = pl.BlockSpec((n, nh), lambda i: (0, 0))
    row_spec = pl.BlockSpec((bm, nh), lambda i: (i, 0))
    vec_spec = pl.BlockSpec((1, nh), lambda i: (0, 0))
    scl_spec = pl.BlockSpec((1, 1), lambda i: (0, 0))
    nin_spec = pl.BlockSpec((n, nin), lambda i: (0, 0))
    w_spec = pl.BlockSpec((nh, nin), lambda i: (0, 0))

    za, sf2 = pl.pallas_call(
        _gcn_a_body,
        grid=(nblk,),
        in_specs=[adj_spec, nin_spec, nin_spec, w_spec, w_spec,
                  vec_spec, scl_spec],
        out_specs=[row_spec, full_spec],
        out_shape=[jax.ShapeDtypeStruct((n, nh), jnp.float32),
                   jax.ShapeDtypeStruct((n, nh), jnp.float32)],
        scratch_shapes=[pltpu.VMEM((n, nh), jnp.float32)],
    )(A, x1, x2, W1, W2, b1r, a1r)

    def fast_two_hop(_):
        c = lambda f: pl.BlockSpec((bm2, n), f)
        d_map = lambda s: (jnp.minimum(s, nblk2 - 1), 0)
        g_map = lambda s: (jnp.where(s < nblk2, 0,
                                     jnp.where(s < 2 * nblk2, s - nblk2,
                                               s - 2 * nblk2)), 0)
        loss_sum = pl.pallas_call(
            _fused_b_body,
            grid=(3 * nblk2,),
            in_specs=[c(d_map), c(g_map), full_spec, vec_spec, scl_spec,
                      full_spec],
            out_specs=pl.BlockSpec((1, 1), lambda s: (0, 0)),
            out_shape=jax.ShapeDtypeStruct((1, 1), jnp.float32),
            scratch_shapes=[pltpu.VMEM((n, nh), jnp.float32),
                            pltpu.VMEM((n, nh), jnp.float32),
                            pltpu.VMEM((n, 1), jnp.float32),
                            pltpu.VMEM((n, 1), jnp.float32)],
            compiler_params=pltpu.CompilerParams(
                vmem_limit_bytes=63 * 1024 * 1024),
        )(D, G, sf2, b2r, a2r, za)
        return loss_sum[0, 0]

    def general_hops(_):
        b_emb = pl.pallas_call(
            _gcn_b_plain_body,
            grid=(nblk,),
            in_specs=[adj_spec, full_spec, vec_spec, scl_spec],
            out_specs=row_spec,
            out_shape=jax.ShapeDtypeStruct((n, nh), jnp.float32),
        )(D, sf2, b2r, a2r)

        def hop(_, x):
            return pl.pallas_call(
                _hop_body,
                grid=(nblk,),
                in_specs=[adj_spec, full_spec],
                out_specs=row_spec,
                out_shape=jax.ShapeDtypeStruct((n, nh), jnp.float32),
            )(G, x)

        nb = jax.lax.fori_loop(0, num_hop, hop, b_emb)
        loss_sum = pl.pallas_call(
            _loss_tail_body,
            grid=(nblk,),
            in_specs=[row_spec, row_spec, full_spec],
            out_specs=pl.BlockSpec((1, 1), lambda i: (0, 0)),
            out_shape=jax.ShapeDtypeStruct((1, 1), jnp.float32),
        )(za, nb, za)
        return loss_sum[0, 0]

    loss_sum = jax.lax.cond(num_hop == 2, fast_two_hop, general_hops, 0)
    return loss_sum / n


# speedup vs baseline: 1.0776x; 1.0776x over previous
import jax
import jax.numpy as jnp
from jax.experimental import pallas as pl
from jax.experimental.pallas import tpu as pltpu

_BM = 400


def _bf16_mm(a, b, dims):
    return jax.lax.dot_general(
        a.astype(jnp.bfloat16), b.astype(jnp.bfloat16), (dims, ((), ())),
        preferred_element_type=jnp.float32)


def _intra_partial(za_ref, i, bm, c0, c1):
    z = za_ref[pl.ds(i * bm, bm), :]
    sim = _bf16_mm(z, za_ref[c0:c1, :], ((1,), (1,)))
    return jnp.sum(jnp.exp(sim), axis=1, keepdims=True)


def _gcn_a_body(adj_ref, bf1_ref, bf2_ref, w1_ref, w2_ref,
                b_ref, a_ref, za_ref, sf2_ref, sf1_ref):
    @pl.when(pl.program_id(0) == 0)
    def _():
        sf1_ref[...] = jax.lax.dot_general(
            bf1_ref[...], w1_ref[...], (((1,), (1,)), ((), ())),
            preferred_element_type=jnp.float32)
        sf2_ref[...] = jax.lax.dot_general(
            bf2_ref[...], w2_ref[...], (((1,), (1,)), ((), ())),
            preferred_element_type=jnp.float32)

    acc = _bf16_mm(adj_ref[...], sf1_ref[...], ((1,), (0,)))
    out = acc + b_ref[...]
    a = a_ref[0, 0]
    out = jnp.where(out >= 0, out, a * out)
    nrm = jnp.sqrt(jnp.sum(out * out, axis=1, keepdims=True))
    za_ref[...] = out / jnp.maximum(nrm, 1e-12)


def _gcn_b_body(adj_ref, sf_ref, b_ref, a_ref, za_ref, out_ref, s_ref):
    acc = _bf16_mm(adj_ref[...], sf_ref[...], ((1,), (0,)))
    out = acc + b_ref[...]
    a = a_ref[0, 0]
    out_ref[...] = jnp.where(out >= 0, out, a * out)
    n = za_ref.shape[0]
    s_ref[...] = _intra_partial(za_ref, pl.program_id(0), out_ref.shape[0],
                                0, n // 2)


def _hop2_body(adj_ref, bemb_ref, za_ref, s1_ref, out_ref, s2_ref, h_ref):
    s = pl.program_id(0)
    nblk = pl.num_programs(0) // 2
    n, nh = h_ref.shape
    bm = adj_ref.shape[0]
    i = s % nblk

    @pl.when(s < nblk)
    def _():
        h_ref[pl.ds(i * bm, bm), :] = _bf16_mm(
            adj_ref[...], bemb_ref[...], ((1,), (0,)))
        s2_ref[pl.ds(i * bm, bm), :] = _intra_partial(za_ref, i, bm, n // 2, n)

    @pl.when(s >= nblk)
    def _():
        nb = _bf16_mm(adj_ref[...], h_ref[...], ((1,), (0,)))
        nrm = jnp.sqrt(jnp.sum(nb * nb, axis=1, keepdims=True))
        znb = nb / jnp.maximum(nrm, 1e-12)
        z = za_ref[pl.ds(i * bm, bm), :]
        inter = jnp.sum(z * znb, axis=1)
        diag = jnp.sum(z * z, axis=1)
        stot = s1_ref[pl.ds(i * bm, bm), 0] + s2_ref[pl.ds(i * bm, bm), 0]
        li = -inter + jnp.log(stot - jnp.exp(diag))

        @pl.when(s == nblk)
        def _():
            out_ref[...] = jnp.zeros((1, 1), jnp.float32)

        out_ref[...] += jnp.sum(li).reshape(1, 1)


def _hop_body(adj_ref, x_ref, out_ref):
    out_ref[...] = _bf16_mm(adj_ref[...], x_ref[...], ((1,), (0,)))


def _loss_tail_body(za_blk_ref, nb_ref, s1_ref, za_ref, out_ref):
    i = pl.program_id(0)
    n = za_ref.shape[0]
    bm = za_blk_ref.shape[0]
    z = za_blk_ref[...]
    nb = nb_ref[...]
    nrm = jnp.sqrt(jnp.sum(nb * nb, axis=1, keepdims=True))
    znb = nb / jnp.maximum(nrm, 1e-12)
    inter = jnp.sum(z * znb, axis=1)
    diag = jnp.sum(z * z, axis=1)
    s2 = _intra_partial(za_ref, i, bm, n // 2, n)
    stot = s1_ref[...][:, 0] + s2[:, 0]
    li = -inter + jnp.log(stot - jnp.exp(diag))

    @pl.when(i == 0)
    def _():
        out_ref[...] = jnp.zeros((1, 1), jnp.float32)

    out_ref[...] += jnp.sum(li).reshape(1, 1)


def kernel(bf1, bf2, ba, bd, adj, W1, b1, a1, W2, b2, a2, num_hop, sparse):
    n = ba.shape[-1]
    nin = bf1.shape[-1]
    nh = W1.shape[0]
    bm = _BM
    nblk = n // bm

    x1 = bf1.reshape(n, nin)
    x2 = bf2.reshape(n, nin)
    A = ba.reshape(n, n)
    D = bd.reshape(n, n)
    G = adj.reshape(n, n)
    b1r = b1.reshape(1, nh)
    b2r = b2.reshape(1, nh)
    a1r = a1.reshape(1, 1)
    a2r = a2.reshape(1, 1)

    adj_spec = pl.BlockSpec((bm, n), lambda i: (i, 0))
    full_spec = pl.BlockSpec((n, nh), lambda i: (0, 0))
    row_spec = pl.BlockSpec((bm, nh), lambda i: (i, 0))
    vec_spec = pl.BlockSpec((1, nh), lambda i: (0, 0))
    scl_spec = pl.BlockSpec((1, 1), lambda i: (0, 0))
    s_spec = pl.BlockSpec((bm, 1), lambda i: (i, 0))
    sful_spec = pl.BlockSpec((n, 1), lambda i: (0, 0))
    nin_spec = pl.BlockSpec((n, nin), lambda i: (0, 0))
    w_spec = pl.BlockSpec((nh, nin), lambda i: (0, 0))

    za, sf2 = pl.pallas_call(
        _gcn_a_body,
        grid=(nblk,),
        in_specs=[adj_spec, nin_spec, nin_spec, w_spec, w_spec,
                  vec_spec, scl_spec],
        out_specs=[row_spec, full_spec],
        out_shape=[jax.ShapeDtypeStruct((n, nh), jnp.float32),
                   jax.ShapeDtypeStruct((n, nh), jnp.float32)],
        scratch_shapes=[pltpu.VMEM((n, nh), jnp.float32)],
    )(A, x1, x2, W1, W2, b1r, a1r)

    b_emb, s1 = pl.pallas_call(
        _gcn_b_body,
        grid=(nblk,),
        in_specs=[adj_spec, full_spec, vec_spec, scl_spec, full_spec],
        out_specs=[row_spec, s_spec],
        out_shape=[jax.ShapeDtypeStruct((n, nh), jnp.float32),
                   jax.ShapeDtypeStruct((n, 1), jnp.float32)],
    )(D, sf2, b2r, a2r, za)

    def fast_two_hop(_):
        loss_sum = pl.pallas_call(
            _hop2_body,
            grid=(2 * nblk,),
            in_specs=[pl.BlockSpec((bm, n), lambda s: (s % nblk, 0)),
                      full_spec, full_spec, sful_spec],
            out_specs=pl.BlockSpec((1, 1), lambda s: (0, 0)),
            out_shape=jax.ShapeDtypeStruct((1, 1), jnp.float32),
            scratch_shapes=[pltpu.VMEM((n, 1), jnp.float32),
                            pltpu.VMEM((n, nh), jnp.float32)],
        )(G, b_emb, za, s1)
        return loss_sum[0, 0]

    def general_hops(_):
        def hop(_, x):
            return pl.pallas_call(
                _hop_body,
                grid=(nblk,),
                in_specs=[adj_spec, full_spec],
                out_specs=row_spec,
                out_shape=jax.ShapeDtypeStruct((n, nh), jnp.float32),
            )(G, x)

        nb = jax.lax.fori_loop(0, num_hop, hop, b_emb)
        loss_sum = pl.pallas_call(
            _loss_tail_body,
            grid=(nblk,),
            in_specs=[row_spec, row_spec, s_spec, full_spec],
            out_specs=pl.BlockSpec((1, 1), lambda i: (0, 0)),
            out_shape=jax.ShapeDtypeStruct((1, 1), jnp.float32),
        )(za, nb, s1, za)
        return loss_sum[0, 0]

    loss_sum = jax.lax.cond(num_hop == 2, fast_two_hop, general_hops, 0)
    return loss_sum / n
